# baseline (speedup 1.0000x reference)
import functools

import jax
import jax.numpy as jnp
from jax import lax
from jax.experimental import pallas as pl
from jax.experimental.pallas import tpu as pltpu
from jax.experimental.pallas import tpu_sc as plsc

SEQ = 8192
D = 1024
B = 4
NC = 2
NS = 16
NW = NC * NS
ROWS_PER_W = SEQ // NW
R = 32
CHUNKS = ROWS_PER_W // R
CHUNK = R * D
LANES = 16
UNROLL = 8


def _sc_body(in_hbm, pos_hbm, out_hbm, pos_buf, data_buf):
    wid = lax.axis_index("s") * NC + lax.axis_index("c")
    s0 = wid * ROWS_PER_W

    for c in range(CHUNKS):
        p_off = (s0 + c * R) * D
        pltpu.sync_copy(pos_hbm.at[pl.ds(p_off, CHUNK)], pos_buf)
        for b in range(B):
            i_off = b * SEQ * D + p_off
            pltpu.sync_copy(in_hbm.at[pl.ds(i_off, CHUNK)], data_buf)

            def add_body(i, _):
                base = i * (LANES * UNROLL)
                for u in range(UNROLL):
                    o = base + u * LANES
                    data_buf[pl.ds(o, LANES)] = (
                        data_buf[pl.ds(o, LANES)] + pos_buf[pl.ds(o, LANES)]
                    )
                return 0

            lax.fori_loop(0, CHUNK // (LANES * UNROLL), add_body, 0)
            pltpu.sync_copy(data_buf, out_hbm.at[pl.ds(i_off, CHUNK)])


@jax.jit
def kernel(inputs, pos_table):
    in_flat = inputs.reshape(B * SEQ * D)
    pos_flat = pos_table.reshape(SEQ * D)
    sc_call = pl.kernel(
        _sc_body,
        out_type=jax.ShapeDtypeStruct((B * SEQ * D,), jnp.float32),
        mesh=plsc.VectorSubcoreMesh(
            core_axis_name="c", subcore_axis_name="s", num_cores=NC, num_subcores=NS
        ),
        scratch_types=[
            pltpu.VMEM((CHUNK,), jnp.float32),
            pltpu.VMEM((CHUNK,), jnp.float32),
        ],
    )
    out_flat = sc_call(in_flat, pos_flat)
    return out_flat.reshape(B, SEQ, D)


# speedup vs baseline: 1.1798x; 1.1798x over previous
import jax
import jax.numpy as jnp
from jax import lax
from jax.experimental import pallas as pl
from jax.experimental.pallas import tpu as pltpu
from jax.experimental.pallas import tpu_sc as plsc

SEQ = 8192
D = 1024
B = 4
NC = 2
NS = 16
NW = NC * NS
ROWS_PER_W = SEQ // NW
R = 8
NCHUNK = ROWS_PER_W // R
RD = R * D
LANES = 16
U = 4


def _sc_body(in_hbm, pos_hbm, out_hbm,
             pos0, pos1, dat0, dat1, sin0, sin1, sout0, sout1):
    wid = lax.axis_index("s") * NC + lax.axis_index("c")
    s0 = wid * ROWS_PER_W

    slots = ((pos0, dat0, sin0, sout0), (pos1, dat1, sin1, sout1))

    def start_in(c):
        pos_b, dat_b, sem, _ = slots[c % 2]
        p_off = (s0 + c * R) * D
        cps = [pltpu.async_copy(pos_hbm.at[pl.ds(p_off, RD)], pos_b, sem)]
        for b in range(B):
            cps.append(
                pltpu.async_copy(
                    in_hbm.at[pl.ds(b * SEQ * D + p_off, RD)],
                    dat_b.at[pl.ds(b * RD, RD)],
                    sem,
                )
            )
        return cps

    def start_out(c):
        pos_b, dat_b, _, sem = slots[c % 2]
        p_off = (s0 + c * R) * D
        cps = []
        for b in range(B):
            cps.append(
                pltpu.async_copy(
                    dat_b.at[pl.ds(b * RD, RD)],
                    out_hbm.at[pl.ds(b * SEQ * D + p_off, RD)],
                    sem,
                )
            )
        return cps

    def compute(c):
        pos_b, dat_b, _, _ = slots[c % 2]

        def body(j, _):
            base = j * (LANES * U)
            for u in range(U):
                o = base + u * LANES
                p = pos_b[pl.ds(o, LANES)]
                for b in range(B):
                    off = b * RD + o
                    dat_b[pl.ds(off, LANES)] = dat_b[pl.ds(off, LANES)] + p
            return 0

        lax.fori_loop(0, RD // (LANES * U), body, 0)

    in_flight = {}
    out_flight = {}

    in_flight[0] = start_in(0)
    for c in range(NCHUNK):
        if c + 1 < NCHUNK:
            if c - 1 >= 0:
                for cp in out_flight.pop(c - 1):
                    cp.wait()
            in_flight[c + 1] = start_in(c + 1)
        for cp in in_flight.pop(c):
            cp.wait()
        compute(c)
        out_flight[c] = start_out(c)
    for c in sorted(out_flight):
        for cp in out_flight[c]:
            cp.wait()


@jax.jit
def kernel(inputs, pos_table):
    in_flat = inputs.reshape(B * SEQ * D)
    pos_flat = pos_table.reshape(SEQ * D)
    sc_call = pl.kernel(
        _sc_body,
        out_type=jax.ShapeDtypeStruct((B * SEQ * D,), jnp.float32),
        mesh=plsc.VectorSubcoreMesh(
            core_axis_name="c", subcore_axis_name="s", num_cores=NC, num_subcores=NS
        ),
        scratch_types=[
            pltpu.VMEM((RD,), jnp.float32),
            pltpu.VMEM((RD,), jnp.float32),
            pltpu.VMEM((B * RD,), jnp.float32),
            pltpu.VMEM((B * RD,), jnp.float32),
            pltpu.SemaphoreType.DMA,
            pltpu.SemaphoreType.DMA,
            pltpu.SemaphoreType.DMA,
            pltpu.SemaphoreType.DMA,
        ],
    )
    out_flat = sc_call(in_flat, pos_flat)
    return out_flat.reshape(B, SEQ, D)


# speedup vs baseline: 1.9145x; 1.6227x over previous
import jax
import jax.numpy as jnp
from jax import lax
from jax.experimental import pallas as pl
from jax.experimental.pallas import tpu as pltpu
from jax.experimental.pallas import tpu_sc as plsc

SEQ = 8192
D = 1024
B = 4
NC = 2
NS = 16
NW = NC * NS
ROWS_PER_W = SEQ // NW
R = 8
NCHUNK = ROWS_PER_W // R
LANES = 16
U = 4


def _sc_body(in_hbm, pos_hbm, out_hbm,
             pos0, pos1, dat0, dat1, sin0, sin1, sout0, sout1):
    wid = lax.axis_index("s") * NC + lax.axis_index("c")
    s0 = wid * ROWS_PER_W

    slots = ((pos0, dat0, sin0, sout0), (pos1, dat1, sin1, sout1))

    def start_in(c):
        pos_b, dat_b, sem, _ = slots[c % 2]
        row0 = s0 + c * R
        cps = [pltpu.async_copy(pos_hbm.at[pl.ds(row0, R)], pos_b, sem)]
        for b in range(B):
            cps.append(
                pltpu.async_copy(in_hbm.at[b, pl.ds(row0, R)], dat_b.at[b], sem)
            )
        return cps

    def start_out(c):
        _, dat_b, _, sem = slots[c % 2]
        row0 = s0 + c * R
        cps = []
        for b in range(B):
            cps.append(
                pltpu.async_copy(dat_b.at[b], out_hbm.at[b, pl.ds(row0, R)], sem)
            )
        return cps

    def compute(c):
        pos_b, dat_b, _, _ = slots[c % 2]

        def row_body(r, _):
            def col_body(j, _):
                base = j * (LANES * U)
                for u in range(U):
                    o = base + u * LANES
                    p = pos_b[r, pl.ds(o, LANES)]
                    for b in range(B):
                        dat_b[b, r, pl.ds(o, LANES)] = (
                            dat_b[b, r, pl.ds(o, LANES)] + p
                        )
                return 0

            lax.fori_loop(0, D // (LANES * U), col_body, 0)
            return 0

        lax.fori_loop(0, R, row_body, 0)

    in_flight = {}
    out_flight = {}

    in_flight[0] = start_in(0)
    for c in range(NCHUNK):
        if c + 1 < NCHUNK:
            if c - 1 >= 0:
                for cp in out_flight.pop(c - 1):
                    cp.wait()
            in_flight[c + 1] = start_in(c + 1)
        for cp in in_flight.pop(c):
            cp.wait()
        compute(c)
        out_flight[c] = start_out(c)
    for c in sorted(out_flight):
        for cp in out_flight[c]:
            cp.wait()


@jax.jit
def kernel(inputs, pos_table):
    sc_call = pl.kernel(
        _sc_body,
        out_type=jax.ShapeDtypeStruct((B, SEQ, D), jnp.float32),
        mesh=plsc.VectorSubcoreMesh(
            core_axis_name="c", subcore_axis_name="s", num_cores=NC, num_subcores=NS
        ),
        scratch_types=[
            pltpu.VMEM((R, D), jnp.float32),
            pltpu.VMEM((R, D), jnp.float32),
            pltpu.VMEM((B, R, D), jnp.float32),
            pltpu.VMEM((B, R, D), jnp.float32),
            pltpu.SemaphoreType.DMA,
            pltpu.SemaphoreType.DMA,
            pltpu.SemaphoreType.DMA,
            pltpu.SemaphoreType.DMA,
        ],
    )
    return sc_call(inputs, pos_table)
